# baseline (speedup 1.0000x reference)
import functools

import jax
import jax.numpy as jnp
from jax.experimental import pallas as pl

EPS = 1e-7
N_GRAPHS = 64


def _gen_conv(x, edge_index, edge_attr, We, be, W1, b1, g, bt, W2, b2):
    src = edge_index[0]
    dst = edge_index[1]
    n = x.shape[0]
    ea = edge_attr @ We + be
    msg = jax.nn.relu(x[src] + ea) + EPS
    m = jax.ops.segment_max(msg, dst, num_segments=n)
    m = jnp.where(jnp.isfinite(m), m, 0.0)
    w = jnp.exp(msg - m[dst])
    denom = jax.ops.segment_sum(w, dst, num_segments=n)
    aggr = jax.ops.segment_sum(msg * w, dst, num_segments=n) / jnp.maximum(denom, 1e-16)
    out = x + aggr
    h = out @ W1 + b1
    mu = jnp.mean(h, axis=0)
    var = jnp.var(h, axis=0)
    h = (h - mu) / jnp.sqrt(var + 1e-5) * g + bt
    h = jax.nn.relu(h)
    return h @ W2 + b2


def _head_kernel(sums_ref, counts_ref, d1w_ref, d1b_ref, d2w_ref, d2b_ref, out_ref):
    h = sums_ref[...] / jnp.maximum(counts_ref[...], 1.0)
    h = h @ d1w_ref[...] + d1b_ref[...]
    h = h @ d2w_ref[...] + d2b_ref[...]
    m = jnp.max(h, axis=-1, keepdims=True)
    lse = m + jnp.log(jnp.sum(jnp.exp(h - m), axis=-1, keepdims=True))
    out_ref[...] = h - lse


def kernel(x, edge_index, edge_attr, batch,
           c1_We, c1_be, c1_W1, c1_b1, c1_g, c1_bt, c1_W2, c1_b2,
           c2_We, c2_be, c2_W1, c2_b1, c2_g, c2_bt, c2_W2, c2_b2,
           c3_We, c3_be, c3_W1, c3_b1, c3_g, c3_bt, c3_W2, c3_b2,
           d1_W, d1_b, d2_W, d2_b):
    h = jax.nn.relu(_gen_conv(x, edge_index, edge_attr, c1_We, c1_be, c1_W1, c1_b1, c1_g, c1_bt, c1_W2, c1_b2))
    h = jax.nn.relu(_gen_conv(h, edge_index, edge_attr, c2_We, c2_be, c2_W1, c2_b1, c2_g, c2_bt, c2_W2, c2_b2))
    h = jax.nn.relu(_gen_conv(h, edge_index, edge_attr, c3_We, c3_be, c3_W1, c3_b1, c3_g, c3_bt, c3_W2, c3_b2))
    ones = jnp.ones((h.shape[0],), h.dtype)
    counts = jax.ops.segment_sum(ones, batch, num_segments=N_GRAPHS)
    sums = jax.ops.segment_sum(h, batch, num_segments=N_GRAPHS)
    out = pl.pallas_call(
        _head_kernel,
        out_shape=jax.ShapeDtypeStruct((N_GRAPHS, d2_W.shape[1]), jnp.float32),
    )(sums, counts[:, None], d1_W, d1_b[None, :], d2_W, d2_b[None, :])
    return out


# speedup vs baseline: 2.3123x; 2.3123x over previous
import functools

import jax
import jax.numpy as jnp
from jax import lax
from jax.experimental import pallas as pl
from jax.experimental.pallas import tpu as pltpu
from jax.experimental.pallas import tpu_sc as plsc

N = 10000
E = 320000
N_GRAPHS = 64
EPS = 1e-7

NC = 2
NS = 16
CH = 80
EP = E // NS
NCHUNK = EP // CH
NZ = N // CH
NZT = (NZ + NS - 1) // NS


def _sc_edge_body(Dh, x2, srcs, dsts, ea2, sflat, out, isrc, idst, xg, eab,
                  ob, sv, acc, sem):
    G = Dh // 16
    G2 = (2 * Dh) // 16
    c = lax.axis_index("c")
    s = lax.axis_index("s")

    def zrow(i, _):
        for j in range(G2):
            ob[i, pl.ds(16 * j, 16)] = jnp.zeros((16,), jnp.float32)
        return 0
    lax.fori_loop(0, CH, zrow, 0)
    for t in range(NZT):
        zi = s + NS * t

        @pl.when(zi < NZ)
        def _():
            pltpu.sync_copy(ob, acc.at[pl.ds(zi * CH, CH)])

    pltpu.sync_copy(sflat.at[pl.ds(c * Dh, Dh)], sv)
    plsc.subcore_barrier()

    def chunk(k, _):
        base = s * EP + k * CH
        pltpu.sync_copy(srcs.at[pl.ds(base, CH)], isrc)
        pltpu.sync_copy(dsts.at[pl.ds(base, CH)], idst)
        for j in range(CH // 16):
            isrc[pl.ds(16 * j, 16)] = isrc[pl.ds(16 * j, 16)] + c * N
        gat = pltpu.async_copy(x2.at[isrc], xg, sem)
        pltpu.sync_copy(ea2.at[pl.ds(c * E + base, CH)], eab)
        gat.wait()

        def row(i, _):
            for j in range(G):
                xv = xg[i, pl.ds(16 * j, 16)]
                ev = eab[i, pl.ds(16 * j, 16)]
                sj = sv[pl.ds(16 * j, 16)]
                msg = jnp.maximum(xv + ev + EPS, EPS)
                w = jnp.exp(msg - sj)
                ob[i, pl.ds(16 * j, 16)] = w
                ob[i, pl.ds(Dh + 16 * j, 16)] = msg * w
            return 0
        lax.fori_loop(0, CH, row, 0)
        pltpu.sync_copy(ob, acc.at[idst], add=True)
        return 0
    lax.fori_loop(0, NCHUNK, chunk, 0)

    plsc.subcore_barrier()
    for t in range(NZT):
        zi = s + NS * t

        @pl.when(zi < NZ)
        def _():
            pltpu.sync_copy(acc.at[pl.ds(zi * CH, CH)], ob)
            pltpu.sync_copy(ob, out.at[pl.ds(c * N + zi * CH, CH)])


def _make_sc_edge(Dh):
    mesh = plsc.VectorSubcoreMesh(core_axis_name="c", subcore_axis_name="s")
    return functools.partial(
        pl.kernel,
        out_type=jax.ShapeDtypeStruct((NC * N, 2 * Dh), jnp.float32),
        mesh=mesh,
        scratch_types=[
            pltpu.VMEM((CH,), jnp.int32),
            pltpu.VMEM((CH,), jnp.int32),
            pltpu.VMEM((CH, Dh), jnp.float32),
            pltpu.VMEM((CH, Dh), jnp.float32),
            pltpu.VMEM((CH, 2 * Dh), jnp.float32),
            pltpu.VMEM((Dh,), jnp.float32),
            pltpu.VMEM_SHARED((N, 2 * Dh), jnp.float32),
            pltpu.SemaphoreType.DMA,
        ],
        compiler_params=pltpu.CompilerParams(use_tc_tiling_on_sc=False),
    )(functools.partial(_sc_edge_body, Dh))


_sc_edge_64 = _make_sc_edge(64)
_sc_edge_32 = _make_sc_edge(32)


TE = 4000


def _ea_body(attr, Wc, bc, o1, o2, o3):
    ea = jnp.dot(attr[...], Wc[...], preferred_element_type=jnp.float32) + bc[...]
    o1[0] = ea[:, 0:64]
    o1[1] = ea[:, 64:128]
    o2[0] = ea[:, 128:160]
    o2[1] = ea[:, 160:192]
    o3[0] = ea[:, 192:224]
    o3[1] = ea[:, 224:256]


def _ea_all(edge_attr, Wc, bc):
    return pl.pallas_call(
        _ea_body,
        grid=(E // TE,),
        in_specs=[
            pl.BlockSpec((TE, 16), lambda i: (i, 0)),
            pl.BlockSpec((16, 256), lambda i: (0, 0)),
            pl.BlockSpec((1, 256), lambda i: (0, 0)),
        ],
        out_specs=[
            pl.BlockSpec((2, TE, 64), lambda i: (0, i, 0)),
            pl.BlockSpec((2, TE, 32), lambda i: (0, i, 0)),
            pl.BlockSpec((2, TE, 32), lambda i: (0, i, 0)),
        ],
        out_shape=[
            jax.ShapeDtypeStruct((2, E, 64), jnp.float32),
            jax.ShapeDtypeStruct((2, E, 32), jnp.float32),
            jax.ShapeDtypeStruct((2, E, 32), jnp.float32),
        ],
    )(edge_attr, Wc, bc)


TN = 1000
NGRID = N // TN


def _aggr_mlp1_body(acc, xs, W1, b1, h_out, sh_out, sh2_out, sh_s, sh2_s):
    i = pl.program_id(0)
    accb = acc[...]
    Dh = accb.shape[2] // 2
    den = jnp.concatenate([accb[0, :, 0:Dh], accb[1, :, 0:Dh]], axis=1)
    num = jnp.concatenate([accb[0, :, Dh:], accb[1, :, Dh:]], axis=1)
    aggr = num / jnp.maximum(den, 1e-38)
    xsb = xs[...]
    xb = jnp.concatenate([xsb[0], xsb[1]], axis=1)
    out = xb + aggr
    h = jnp.dot(out, W1[...], preferred_element_type=jnp.float32) + b1[...]
    h_out[...] = h

    @pl.when(i == 0)
    def _():
        sh_s[...] = jnp.zeros_like(sh_s)
        sh2_s[...] = jnp.zeros_like(sh2_s)

    sh_s[...] += jnp.sum(h, axis=0, keepdims=True)
    sh2_s[...] += jnp.sum(h * h, axis=0, keepdims=True)

    @pl.when(i == NGRID - 1)
    def _():
        sh_out[...] = sh_s[...]
        sh2_out[...] = sh2_s[...]


def _aggr_mlp1(acc3, xsplit, W1, b1):
    D = W1.shape[0]
    H = W1.shape[1]
    return pl.pallas_call(
        _aggr_mlp1_body,
        grid=(NGRID,),
        in_specs=[
            pl.BlockSpec((2, TN, D), lambda i: (0, i, 0)),
            pl.BlockSpec((2, TN, D // 2), lambda i: (0, i, 0)),
            pl.BlockSpec((D, H), lambda i: (0, 0)),
            pl.BlockSpec((1, H), lambda i: (0, 0)),
        ],
        out_specs=[
            pl.BlockSpec((TN, H), lambda i: (i, 0)),
            pl.BlockSpec((1, H), lambda i: (0, 0)),
            pl.BlockSpec((1, H), lambda i: (0, 0)),
        ],
        out_shape=[
            jax.ShapeDtypeStruct((N, H), jnp.float32),
            jax.ShapeDtypeStruct((1, H), jnp.float32),
            jax.ShapeDtypeStruct((1, H), jnp.float32),
        ],
        scratch_shapes=[
            pltpu.VMEM((1, H), jnp.float32),
            pltpu.VMEM((1, H), jnp.float32),
        ],
    )(acc3, xsplit, W1, b1)


def _bn_mlp2_body(h, sh, sh2, g, bt, W2, b2, y_out, xmax_out, xmax_s):
    i = pl.program_id(0)
    mu = sh[...] / N
    var = sh2[...] / N - mu * mu
    hn = (h[...] - mu) * lax.rsqrt(var + 1e-5) * g[...] + bt[...]
    hn = jnp.maximum(hn, 0.0)
    y = jnp.dot(hn, W2[...], preferred_element_type=jnp.float32) + b2[...]
    y = jnp.maximum(y, 0.0)
    Dh = y.shape[1] // 2
    y_out[0] = y[:, 0:Dh]
    y_out[1] = y[:, Dh:]

    @pl.when(i == 0)
    def _():
        xmax_s[...] = jnp.full_like(xmax_s, -jnp.inf)

    xmax_s[...] = jnp.maximum(xmax_s[...], jnp.max(y, axis=0, keepdims=True))

    @pl.when(i == NGRID - 1)
    def _():
        xmax_out[...] = xmax_s[...]


def _bn_mlp2(h, sh, sh2, g, bt, W2, b2):
    H = W2.shape[0]
    Do = W2.shape[1]
    return pl.pallas_call(
        _bn_mlp2_body,
        grid=(NGRID,),
        in_specs=[
            pl.BlockSpec((TN, H), lambda i: (i, 0)),
            pl.BlockSpec((1, H), lambda i: (0, 0)),
            pl.BlockSpec((1, H), lambda i: (0, 0)),
            pl.BlockSpec((1, H), lambda i: (0, 0)),
            pl.BlockSpec((1, H), lambda i: (0, 0)),
            pl.BlockSpec((H, Do), lambda i: (0, 0)),
            pl.BlockSpec((1, Do), lambda i: (0, 0)),
        ],
        out_specs=[
            pl.BlockSpec((2, TN, Do // 2), lambda i: (0, i, 0)),
            pl.BlockSpec((1, Do), lambda i: (0, 0)),
        ],
        out_shape=[
            jax.ShapeDtypeStruct((2, N, Do // 2), jnp.float32),
            jax.ShapeDtypeStruct((1, Do), jnp.float32),
        ],
        scratch_shapes=[pltpu.VMEM((1, Do), jnp.float32)],
    )(h, sh, sh2, g, bt, W2, b2)


def _pool_head_body(hsplit, batch3, d1W, d1b, d2W, d2b, out, pool_s, cnt_s):
    i = pl.program_id(0)
    b = batch3[...].reshape(1, TN)
    gid = lax.broadcasted_iota(jnp.int32, (N_GRAPHS, TN), 0)
    onehot = jnp.where(gid == b, 1.0, 0.0).astype(jnp.float32)
    hb = hsplit[...]
    hcat = jnp.concatenate([hb[0], hb[1]], axis=1)

    @pl.when(i == 0)
    def _():
        pool_s[...] = jnp.zeros_like(pool_s)
        cnt_s[...] = jnp.zeros_like(cnt_s)

    pool_s[...] += jnp.dot(onehot, hcat, preferred_element_type=jnp.float32)
    cnt_s[...] += jnp.sum(onehot, axis=1, keepdims=True)

    @pl.when(i == NGRID - 1)
    def _():
        mean = pool_s[...] / jnp.maximum(cnt_s[...], 1.0)
        z = jnp.dot(mean, d1W[...], preferred_element_type=jnp.float32) + d1b[...]
        z = jnp.dot(z, d2W[...], preferred_element_type=jnp.float32) + d2b[...]
        m = jnp.max(z, axis=-1, keepdims=True)
        lse = m + jnp.log(jnp.sum(jnp.exp(z - m), axis=-1, keepdims=True))
        out[...] = z - lse


def _pool_head(hsplit, batch3, d1W, d1b, d2W, d2b):
    return pl.pallas_call(
        _pool_head_body,
        grid=(NGRID,),
        in_specs=[
            pl.BlockSpec((2, TN, 64), lambda i: (0, i, 0)),
            pl.BlockSpec((1, 1, TN), lambda i: (i, 0, 0)),
            pl.BlockSpec((128, 64), lambda i: (0, 0)),
            pl.BlockSpec((1, 64), lambda i: (0, 0)),
            pl.BlockSpec((64, 10), lambda i: (0, 0)),
            pl.BlockSpec((1, 10), lambda i: (0, 0)),
        ],
        out_specs=pl.BlockSpec((N_GRAPHS, 10), lambda i: (0, 0)),
        out_shape=jax.ShapeDtypeStruct((N_GRAPHS, 10), jnp.float32),
        scratch_shapes=[
            pltpu.VMEM((N_GRAPHS, 128), jnp.float32),
            pltpu.VMEM((N_GRAPHS, 1), jnp.float32),
        ],
    )(hsplit, batch3, d1W, d1b, d2W, d2b)


def _layer(xsplit, xmax, srcs, dsts, ea2, colmax, We, be, W1, b1, g, bt, W2, b2):
    D = W1.shape[0]
    Dh = D // 2
    eabound = jnp.abs(We).T @ colmax + be
    S = jnp.maximum(xmax + eabound, 0.0) + EPS
    sc = _sc_edge_64 if Dh == 64 else _sc_edge_32
    acc = sc(xsplit.reshape(2 * N, Dh), srcs, dsts, ea2.reshape(2 * E, Dh), S)
    acc3 = acc.reshape(2, N, 2 * Dh)
    h, sh, sh2 = _aggr_mlp1(acc3, xsplit, W1, b1[None, :])
    return _bn_mlp2(h, sh, sh2, g[None, :], bt[None, :], W2, b2[None, :])


def kernel(x, edge_index, edge_attr, batch,
           c1_We, c1_be, c1_W1, c1_b1, c1_g, c1_bt, c1_W2, c1_b2,
           c2_We, c2_be, c2_W1, c2_b1, c2_g, c2_bt, c2_W2, c2_b2,
           c3_We, c3_be, c3_W1, c3_b1, c3_g, c3_bt, c3_W2, c3_b2,
           d1_W, d1_b, d2_W, d2_b):
    srcs = edge_index[0]
    dsts = edge_index[1]
    Wc = jnp.concatenate([c1_We, c2_We, c3_We], axis=1)
    bc = jnp.concatenate([c1_be, c2_be, c3_be])[None, :]
    ea1, ea2, ea3 = _ea_all(edge_attr, Wc, bc)
    colmax = jnp.max(jnp.abs(edge_attr), axis=0)

    x1 = x.reshape(N, 2, 64).transpose(1, 0, 2)
    xmax1 = jnp.max(x, axis=0)
    x2s, xmax2 = _layer(x1, xmax1, srcs, dsts, ea1, colmax,
                        c1_We, c1_be, c1_W1, c1_b1, c1_g, c1_bt, c1_W2, c1_b2)
    x3s, xmax3 = _layer(x2s, xmax2.reshape(-1), srcs, dsts, ea2, colmax,
                        c2_We, c2_be, c2_W1, c2_b1, c2_g, c2_bt, c2_W2, c2_b2)
    x4s, _ = _layer(x3s, xmax3.reshape(-1), srcs, dsts, ea3, colmax,
                    c3_We, c3_be, c3_W1, c3_b1, c3_g, c3_bt, c3_W2, c3_b2)

    batch3 = batch.reshape(NGRID, 1, TN)
    return _pool_head(x4s, batch3, d1_W, d1_b[None, :], d2_W, d2_b[None, :])


# speedup vs baseline: 3.3194x; 1.4356x over previous
import functools

import jax
import jax.numpy as jnp
from jax import lax
from jax.experimental import pallas as pl
from jax.experimental.pallas import tpu as pltpu
from jax.experimental.pallas import tpu_sc as plsc

N = 10000
E = 320000
N_GRAPHS = 64
EPS = 1e-7

NC = 2
NS = 16
EP = E // NS


def _sc_edge_body(Dh, CH, NB, x2, srcs2, dsts, ea2, sflat, out, isrc, idst,
                  xg, eab, ob, sv, acc, sem_i, sem_e, sem_g, sem_s):
    NCHUNK = EP // CH
    NSUP = NCHUNK // NB
    NZ = N // CH
    NZT = (NZ + NS - 1) // NS
    G = Dh // 16
    G2 = (2 * Dh) // 16
    c = lax.axis_index("c")
    s = lax.axis_index("s")

    def zrow(i, _):
        for j in range(G2):
            ob[0, i, pl.ds(16 * j, 16)] = jnp.zeros((16,), jnp.float32)
        return 0
    lax.fori_loop(0, CH, zrow, 0)
    for t in range(NZT):
        zi = s + NS * t

        @pl.when(zi < NZ)
        def _():
            pltpu.sync_copy(ob.at[0], acc.at[pl.ds(zi * CH, CH)])

    pltpu.sync_copy(sflat.at[pl.ds(c * Dh, Dh)], sv)
    svs = tuple(sv[pl.ds(16 * j, 16)] for j in range(G))
    plsc.subcore_barrier()

    def start_idx(k, b):
        base = s * EP + k * CH
        pltpu.async_copy(srcs2.at[pl.ds(c * E + base, CH)], isrc.at[b], sem_i.at[b])
        pltpu.async_copy(dsts.at[pl.ds(base, CH)], idst.at[b], sem_i.at[b])
        pltpu.async_copy(ea2.at[pl.ds(c * E + base, CH)], eab.at[b], sem_e.at[b])

    def wait_idx(b):
        pltpu.make_async_copy(dsts.at[pl.ds(0, CH)], isrc.at[b], sem_i.at[b]).wait()
        pltpu.make_async_copy(dsts.at[pl.ds(0, CH)], idst.at[b], sem_i.at[b]).wait()

    def start_gather(b):
        pltpu.async_copy(x2.at[isrc.at[b]], xg.at[b], sem_g.at[b])

    def wait_gather_ea(b):
        pltpu.make_async_copy(x2.at[isrc.at[b]], xg.at[b], sem_g.at[b]).wait()
        pltpu.make_async_copy(ea2.at[pl.ds(0, CH)], eab.at[b], sem_e.at[b]).wait()

    def start_scatter(b):
        pltpu.async_copy(ob.at[b], acc.at[idst.at[b]], sem_s.at[b], add=True)

    def wait_scatter(b):
        pltpu.make_async_copy(ob.at[b], acc.at[idst.at[b]], sem_s.at[b]).wait()

    start_idx(0, 0)
    start_idx(1, 1)
    wait_idx(0)
    start_gather(0)

    def super_chunk(k5, carry):
        svs = carry
        for b in range(NB):
            k = k5 * NB + b

            @pl.when(k >= 2)
            def _():
                wait_scatter((b - 2) % NB)

            @pl.when(k + 2 < NCHUNK)
            def _():
                start_idx(k + 2, (b + 2) % NB)

            @pl.when(k + 1 < NCHUNK)
            def _():
                wait_idx((b + 1) % NB)
                start_gather((b + 1) % NB)

            wait_gather_ea(b)

            def row(i, _):
                for r in range(2):
                    for j in range(G):
                        xv = xg[b, 2 * i + r, pl.ds(16 * j, 16)]
                        ev = eab[b, 2 * i + r, pl.ds(16 * j, 16)]
                        msg = jnp.maximum(xv + ev + EPS, EPS)
                        w = jnp.exp(msg - svs[j])
                        ob[b, 2 * i + r, pl.ds(16 * j, 16)] = w
                        ob[b, 2 * i + r, pl.ds(Dh + 16 * j, 16)] = msg * w
                return 0
            lax.fori_loop(0, CH // 2, row, 0)
            start_scatter(b)
        return svs
    lax.fori_loop(0, NSUP, super_chunk, svs)
    wait_scatter((NCHUNK - 2) % NB)
    wait_scatter((NCHUNK - 1) % NB)

    plsc.subcore_barrier()
    for t in range(NZT):
        zi = s + NS * t

        @pl.when(zi < NZ)
        def _():
            pltpu.sync_copy(acc.at[pl.ds(zi * CH, CH)], ob.at[0])
            pltpu.sync_copy(ob.at[0], out.at[pl.ds(c * N + zi * CH, CH)])


def _make_sc_edge(Dh, CH, NB):
    mesh = plsc.VectorSubcoreMesh(core_axis_name="c", subcore_axis_name="s")
    return functools.partial(
        pl.kernel,
        out_type=jax.ShapeDtypeStruct((NC * N, 2 * Dh), jnp.float32),
        mesh=mesh,
        scratch_types=[
            pltpu.VMEM((NB, CH), jnp.int32),
            pltpu.VMEM((NB, CH), jnp.int32),
            pltpu.VMEM((NB, CH, Dh), jnp.float32),
            pltpu.VMEM((NB, CH, Dh), jnp.float32),
            pltpu.VMEM((NB, CH, 2 * Dh), jnp.float32),
            pltpu.VMEM((Dh,), jnp.float32),
            pltpu.VMEM_SHARED((N, 2 * Dh), jnp.float32),
            pltpu.SemaphoreType.DMA((NB,)),
            pltpu.SemaphoreType.DMA((NB,)),
            pltpu.SemaphoreType.DMA((NB,)),
            pltpu.SemaphoreType.DMA((NB,)),
        ],
        compiler_params=pltpu.CompilerParams(use_tc_tiling_on_sc=False),
    )(functools.partial(_sc_edge_body, Dh, CH, NB))


_sc_edge_64 = _make_sc_edge(64, 40, 4)
_sc_edge_32 = _make_sc_edge(32, 80, 5)


TE = 4000


def _ea_body(attr, Wc, bc, o1, o2, o3):
    ea = jnp.dot(attr[...], Wc[...], preferred_element_type=jnp.float32) + bc[...]
    o1[0] = ea[:, 0:64]
    o1[1] = ea[:, 64:128]
    o2[0] = ea[:, 128:160]
    o2[1] = ea[:, 160:192]
    o3[0] = ea[:, 192:224]
    o3[1] = ea[:, 224:256]


def _ea_all(edge_attr, Wc, bc):
    return pl.pallas_call(
        _ea_body,
        grid=(E // TE,),
        in_specs=[
            pl.BlockSpec((TE, 16), lambda i: (i, 0)),
            pl.BlockSpec((16, 256), lambda i: (0, 0)),
            pl.BlockSpec((1, 256), lambda i: (0, 0)),
        ],
        out_specs=[
            pl.BlockSpec((2, TE, 64), lambda i: (0, i, 0)),
            pl.BlockSpec((2, TE, 32), lambda i: (0, i, 0)),
            pl.BlockSpec((2, TE, 32), lambda i: (0, i, 0)),
        ],
        out_shape=[
            jax.ShapeDtypeStruct((2, E, 64), jnp.float32),
            jax.ShapeDtypeStruct((2, E, 32), jnp.float32),
            jax.ShapeDtypeStruct((2, E, 32), jnp.float32),
        ],
    )(edge_attr, Wc, bc)


TN = 1000
NGRID = N // TN


def _aggr_mlp1_body(acc, xs, W1, b1, h_out, sh_out, sh2_out, sh_s, sh2_s):
    i = pl.program_id(0)
    accb = acc[...]
    Dh = accb.shape[2] // 2
    den = jnp.concatenate([accb[0, :, 0:Dh], accb[1, :, 0:Dh]], axis=1)
    num = jnp.concatenate([accb[0, :, Dh:], accb[1, :, Dh:]], axis=1)
    aggr = num / jnp.maximum(den, 1e-38)
    xsb = xs[...]
    xb = jnp.concatenate([xsb[0], xsb[1]], axis=1)
    out = xb + aggr
    h = jnp.dot(out, W1[...], preferred_element_type=jnp.float32) + b1[...]
    h_out[...] = h

    @pl.when(i == 0)
    def _():
        sh_s[...] = jnp.zeros_like(sh_s)
        sh2_s[...] = jnp.zeros_like(sh2_s)

    sh_s[...] += jnp.sum(h, axis=0, keepdims=True)
    sh2_s[...] += jnp.sum(h * h, axis=0, keepdims=True)

    @pl.when(i == NGRID - 1)
    def _():
        sh_out[...] = sh_s[...]
        sh2_out[...] = sh2_s[...]


def _aggr_mlp1(acc3, xsplit, W1, b1):
    D = W1.shape[0]
    H = W1.shape[1]
    return pl.pallas_call(
        _aggr_mlp1_body,
        grid=(NGRID,),
        in_specs=[
            pl.BlockSpec((2, TN, D), lambda i: (0, i, 0)),
            pl.BlockSpec((2, TN, D // 2), lambda i: (0, i, 0)),
            pl.BlockSpec((D, H), lambda i: (0, 0)),
            pl.BlockSpec((1, H), lambda i: (0, 0)),
        ],
        out_specs=[
            pl.BlockSpec((TN, H), lambda i: (i, 0)),
            pl.BlockSpec((1, H), lambda i: (0, 0)),
            pl.BlockSpec((1, H), lambda i: (0, 0)),
        ],
        out_shape=[
            jax.ShapeDtypeStruct((N, H), jnp.float32),
            jax.ShapeDtypeStruct((1, H), jnp.float32),
            jax.ShapeDtypeStruct((1, H), jnp.float32),
        ],
        scratch_shapes=[
            pltpu.VMEM((1, H), jnp.float32),
            pltpu.VMEM((1, H), jnp.float32),
        ],
    )(acc3, xsplit, W1, b1)


def _bn_mlp2_body(h, sh, sh2, g, bt, W2, b2, y_out, xmax_out, xmax_s):
    i = pl.program_id(0)
    mu = sh[...] / N
    var = sh2[...] / N - mu * mu
    hn = (h[...] - mu) * lax.rsqrt(var + 1e-5) * g[...] + bt[...]
    hn = jnp.maximum(hn, 0.0)
    y = jnp.dot(hn, W2[...], preferred_element_type=jnp.float32) + b2[...]
    y = jnp.maximum(y, 0.0)
    Dh = y.shape[1] // 2
    y_out[0] = y[:, 0:Dh]
    y_out[1] = y[:, Dh:]

    @pl.when(i == 0)
    def _():
        xmax_s[...] = jnp.full_like(xmax_s, -jnp.inf)

    xmax_s[...] = jnp.maximum(xmax_s[...], jnp.max(y, axis=0, keepdims=True))

    @pl.when(i == NGRID - 1)
    def _():
        xmax_out[...] = xmax_s[...]


def _bn_mlp2(h, sh, sh2, g, bt, W2, b2):
    H = W2.shape[0]
    Do = W2.shape[1]
    return pl.pallas_call(
        _bn_mlp2_body,
        grid=(NGRID,),
        in_specs=[
            pl.BlockSpec((TN, H), lambda i: (i, 0)),
            pl.BlockSpec((1, H), lambda i: (0, 0)),
            pl.BlockSpec((1, H), lambda i: (0, 0)),
            pl.BlockSpec((1, H), lambda i: (0, 0)),
            pl.BlockSpec((1, H), lambda i: (0, 0)),
            pl.BlockSpec((H, Do), lambda i: (0, 0)),
            pl.BlockSpec((1, Do), lambda i: (0, 0)),
        ],
        out_specs=[
            pl.BlockSpec((2, TN, Do // 2), lambda i: (0, i, 0)),
            pl.BlockSpec((1, Do), lambda i: (0, 0)),
        ],
        out_shape=[
            jax.ShapeDtypeStruct((2, N, Do // 2), jnp.float32),
            jax.ShapeDtypeStruct((1, Do), jnp.float32),
        ],
        scratch_shapes=[pltpu.VMEM((1, Do), jnp.float32)],
    )(h, sh, sh2, g, bt, W2, b2)


def _pool_head_body(hsplit, batch3, d1W, d1b, d2W, d2b, out, pool_s, cnt_s):
    i = pl.program_id(0)
    b = batch3[...].reshape(1, TN)
    gid = lax.broadcasted_iota(jnp.int32, (N_GRAPHS, TN), 0)
    onehot = jnp.where(gid == b, 1.0, 0.0).astype(jnp.float32)
    hb = hsplit[...]
    hcat = jnp.concatenate([hb[0], hb[1]], axis=1)

    @pl.when(i == 0)
    def _():
        pool_s[...] = jnp.zeros_like(pool_s)
        cnt_s[...] = jnp.zeros_like(cnt_s)

    pool_s[...] += jnp.dot(onehot, hcat, preferred_element_type=jnp.float32)
    cnt_s[...] += jnp.sum(onehot, axis=1, keepdims=True)

    @pl.when(i == NGRID - 1)
    def _():
        mean = pool_s[...] / jnp.maximum(cnt_s[...], 1.0)
        z = jnp.dot(mean, d1W[...], preferred_element_type=jnp.float32) + d1b[...]
        z = jnp.dot(z, d2W[...], preferred_element_type=jnp.float32) + d2b[...]
        m = jnp.max(z, axis=-1, keepdims=True)
        lse = m + jnp.log(jnp.sum(jnp.exp(z - m), axis=-1, keepdims=True))
        out[...] = z - lse


def _pool_head(hsplit, batch3, d1W, d1b, d2W, d2b):
    return pl.pallas_call(
        _pool_head_body,
        grid=(NGRID,),
        in_specs=[
            pl.BlockSpec((2, TN, 64), lambda i: (0, i, 0)),
            pl.BlockSpec((1, 1, TN), lambda i: (i, 0, 0)),
            pl.BlockSpec((128, 64), lambda i: (0, 0)),
            pl.BlockSpec((1, 64), lambda i: (0, 0)),
            pl.BlockSpec((64, 10), lambda i: (0, 0)),
            pl.BlockSpec((1, 10), lambda i: (0, 0)),
        ],
        out_specs=pl.BlockSpec((N_GRAPHS, 10), lambda i: (0, 0)),
        out_shape=jax.ShapeDtypeStruct((N_GRAPHS, 10), jnp.float32),
        scratch_shapes=[
            pltpu.VMEM((N_GRAPHS, 128), jnp.float32),
            pltpu.VMEM((N_GRAPHS, 1), jnp.float32),
        ],
    )(hsplit, batch3, d1W, d1b, d2W, d2b)


def _layer(xsplit, xmax, srcs2, dsts, ea2, colmax, We, be, W1, b1, g, bt, W2, b2):
    D = W1.shape[0]
    Dh = D // 2
    eabound = jnp.abs(We).T @ colmax + be
    S = jnp.maximum(xmax + eabound, 0.0) + EPS
    sc = _sc_edge_64 if Dh == 64 else _sc_edge_32
    acc = sc(xsplit.reshape(2 * N, Dh), srcs2, dsts, ea2.reshape(2 * E, Dh), S)
    acc3 = acc.reshape(2, N, 2 * Dh)
    h, sh, sh2 = _aggr_mlp1(acc3, xsplit, W1, b1[None, :])
    return _bn_mlp2(h, sh, sh2, g[None, :], bt[None, :], W2, b2[None, :])


def kernel(x, edge_index, edge_attr, batch,
           c1_We, c1_be, c1_W1, c1_b1, c1_g, c1_bt, c1_W2, c1_b2,
           c2_We, c2_be, c2_W1, c2_b1, c2_g, c2_bt, c2_W2, c2_b2,
           c3_We, c3_be, c3_W1, c3_b1, c3_g, c3_bt, c3_W2, c3_b2,
           d1_W, d1_b, d2_W, d2_b):
    srcs = edge_index[0]
    dsts = edge_index[1]
    srcs2 = jnp.concatenate([srcs, srcs + N])
    Wc = jnp.concatenate([c1_We, c2_We, c3_We], axis=1)
    bc = jnp.concatenate([c1_be, c2_be, c3_be])[None, :]
    ea1, ea2, ea3 = _ea_all(edge_attr, Wc, bc)
    colmax = jnp.max(jnp.abs(edge_attr), axis=0)

    x1 = x.reshape(N, 2, 64).transpose(1, 0, 2)
    xmax1 = jnp.max(x, axis=0)
    x2s, xmax2 = _layer(x1, xmax1, srcs2, dsts, ea1, colmax,
                        c1_We, c1_be, c1_W1, c1_b1, c1_g, c1_bt, c1_W2, c1_b2)
    x3s, xmax3 = _layer(x2s, xmax2.reshape(-1), srcs2, dsts, ea2, colmax,
                        c2_We, c2_be, c2_W1, c2_b1, c2_g, c2_bt, c2_W2, c2_b2)
    x4s, _ = _layer(x3s, xmax3.reshape(-1), srcs2, dsts, ea3, colmax,
                    c3_We, c3_be, c3_W1, c3_b1, c3_g, c3_bt, c3_W2, c3_b2)

    batch3 = batch.reshape(NGRID, 1, TN)
    return _pool_head(x4s, batch3, d1_W, d1_b[None, :], d2_W, d2_b[None, :])
